# Initial kernel scaffold; baseline (speedup 1.0000x reference)
import jax
import jax.numpy as jnp
from jax.experimental import pallas as pl


def kernel(feats, edge_index, edge_attr, W_fc, W_edge, att, bias, W_out, b_out):
    raise NotImplementedError("write your pallas kernel here")


# speedup vs baseline: 15.6761x; 15.6761x over previous
import functools

import jax
import jax.numpy as jnp
from jax import lax
from jax.experimental import pallas as pl
from jax.experimental.pallas import tpu as pltpu
from jax.experimental.pallas import tpu_sc as plsc

N = 10000
E = 160000
D = 256
H = 8
HD = 32

NP = 10240
EP = 163840
NBLK = EP // 128
STRIPE = NP // 16

f32 = jnp.float32
i32 = jnp.int32

_MESH = plsc.VectorSubcoreMesh(core_axis_name="c", subcore_axis_name="s")
_SC_PARAMS = pltpu.CompilerParams(
    needs_layout_passes=False, use_tc_tiling_on_sc=False
)


def _node_body(x_ref, wt_ref, a12_ref, h2_ref, src_ref):
    t = pl.program_id(0)
    x = x_ref[...]
    h = jnp.dot(x, wt_ref[...], preferred_element_type=f32)
    rows = t * 1024 + lax.broadcasted_iota(i32, (1024, 1), 0)
    h = jnp.where(rows < N, h, 0.0)
    h2_ref[0, :, :] = h[:, :128]
    h2_ref[1, :, :] = h[:, 128:]
    src_ref[...] = jnp.dot(h, a12_ref[...], preferred_element_type=f32)


def _k_node(feats, wt, a12):
    return pl.pallas_call(
        _node_body,
        grid=(10,),
        in_specs=[
            pl.BlockSpec((1024, 256), lambda t: (t, 0)),
            pl.BlockSpec((256, 256), lambda t: (0, 0)),
            pl.BlockSpec((256, 16), lambda t: (0, 0)),
        ],
        out_specs=[
            pl.BlockSpec((2, 1024, 128), lambda t: (0, t, 0)),
            pl.BlockSpec((1024, 16), lambda t: (t, 0)),
        ],
        out_shape=[
            jax.ShapeDtypeStruct((2, NP, 128), f32),
            jax.ShapeDtypeStruct((NP, 16), f32),
        ],
    )(feats, wt, a12)


def _edge_body(x_ref, wt_ref, b8_ref, ea_ref, t_ref):
    ea = jnp.dot(x_ref[...], wt_ref[...], preferred_element_type=f32)
    ea_ref[...] = ea
    t_ref[...] = jnp.dot(ea, b8_ref[...], preferred_element_type=f32)


def _k_edge(edge_attr, wt, b8):
    return pl.pallas_call(
        _edge_body,
        grid=(80,),
        in_specs=[
            pl.BlockSpec((2000, 16), lambda t: (t, 0)),
            pl.BlockSpec((16, 16), lambda t: (0, 0)),
            pl.BlockSpec((16, 8), lambda t: (0, 0)),
        ],
        out_specs=[
            pl.BlockSpec((2000, 16), lambda t: (t, 0)),
            pl.BlockSpec((2000, 8), lambda t: (t, 0)),
        ],
        out_shape=[
            jax.ShapeDtypeStruct((E, 16), f32),
            jax.ShapeDtypeStruct((E, 8), f32),
        ],
    )(edge_attr, wt, b8)


def _rden_body(dp_ref, r_ref):
    r_ref[...] = 1.0 / (dp_ref[0, :, :] + dp_ref[1, :, :])


def _k_rden(dp3):
    return pl.pallas_call(
        _rden_body,
        out_shape=jax.ShapeDtypeStruct((NP * 8 // 128, 128), f32),
    )(dp3)


def _out_body(a0_ref, a1_ref, w0_ref, w1_ref, blo_ref, bhi_ref, bo_ref, o_ref):
    a0 = a0_ref[...] + blo_ref[...]
    a1 = a1_ref[...] + bhi_ref[...]
    o_ref[...] = (
        jnp.dot(a0, w0_ref[...], preferred_element_type=f32)
        + jnp.dot(a1, w1_ref[...], preferred_element_type=f32)
        + bo_ref[...]
    )


def _k_out(a0, a1, w0t, w1t, blo, bhi, bo):
    return pl.pallas_call(
        _out_body,
        grid=(10,),
        in_specs=[
            pl.BlockSpec((1024, 128), lambda t: (t, 0)),
            pl.BlockSpec((1024, 128), lambda t: (t, 0)),
            pl.BlockSpec((128, 256), lambda t: (0, 0)),
            pl.BlockSpec((128, 256), lambda t: (0, 0)),
            pl.BlockSpec((1, 128), lambda t: (0, 0)),
            pl.BlockSpec((1, 128), lambda t: (0, 0)),
            pl.BlockSpec((1, 256), lambda t: (0, 0)),
        ],
        out_specs=pl.BlockSpec((1024, 256), lambda t: (t, 0)),
        out_shape=jax.ShapeDtypeStruct((N, 256), f32),
    )(a0, a1, w0t, w1t, blo, bhi, bo)


def _attn_body(row_hbm, col_hbm, t_hbm, src_hbm, z8_hbm, ex_hbm, dp_hbm,
               rowb, colb, srg, scg, tbuf, exflat, ex2d, acc):
    c = lax.axis_index("c")
    s = lax.axis_index("s")
    w = c * 16 + s
    pltpu.sync_copy(z8_hbm.at[pl.ds(s * STRIPE, STRIPE), :],
                    acc.at[pl.ds(s * STRIPE, STRIPE), :])
    plsc.subcore_barrier()

    iota = lax.iota(i32, 16)
    id8 = iota // 8
    im8 = iota % 8

    @pl.loop(0, NBLK // 32)
    def _blk(k):
        g = w * (NBLK // 32) + k
        pltpu.sync_copy(row_hbm.at[pl.ds(g, 1), :], rowb)
        pltpu.sync_copy(col_hbm.at[pl.ds(g, 1), :], colb)
        pltpu.sync_copy(t_hbm.at[pl.ds(g * 1024, 1024)], tbuf)
        pltpu.sync_copy(src_hbm.at[rowb.at[0]], srg)
        pltpu.sync_copy(src_hbm.at[colb.at[0]], scg)

        @pl.loop(0, 64)
        def _vec(v):
            ir = 2 * v + id8
            srv = plsc.load_gather(srg, [ir, im8])
            scv = plsc.load_gather(scg, [ir, im8 + 8])
            tv = tbuf[pl.ds(16 * v, 16)]
            lg = srv + scv + tv
            exv = jnp.exp(jnp.maximum(lg, 0.01 * lg))
            exflat[pl.ds(16 * v, 16)] = exv
            plsc.store_scatter(ex2d, [ir, im8], exv)

        pltpu.sync_copy(ex2d, acc.at[rowb.at[0]], add=True)
        pltpu.sync_copy(exflat, ex_hbm.at[pl.ds(g * 1024, 1024)])

    plsc.subcore_barrier()
    pltpu.sync_copy(acc.at[pl.ds(s * STRIPE, STRIPE), :],
                    dp_hbm.at[c].at[pl.ds(s * STRIPE, STRIPE), :])


def _k_attn(row2d, col2d, tp, src_tab, z8):
    return pl.kernel(
        _attn_body,
        out_type=(
            jax.ShapeDtypeStruct((EP * 8,), f32),
            jax.ShapeDtypeStruct((2, NP, 8), f32),
        ),
        mesh=_MESH,
        compiler_params=_SC_PARAMS,
        scratch_types=[
            pltpu.VMEM((1, 128), i32),
            pltpu.VMEM((1, 128), i32),
            pltpu.VMEM((128, 16), f32),
            pltpu.VMEM((128, 16), f32),
            pltpu.VMEM((1024,), f32),
            pltpu.VMEM((1024,), f32),
            pltpu.VMEM((128, 8), f32),
            pltpu.VMEM_SHARED((NP, 8), f32),
        ],
    )(row2d, col2d, tp, src_tab, z8)


def _agg_body(row_hbm, col_hbm, ex_hbm, rden_hbm, h2_hbm, z128_hbm, agg_hbm,
              rowb, colb, colb2, hbuf, rdbuf, exbuf, albuf, acc):
    c = lax.axis_index("c")
    s = lax.axis_index("s")
    pltpu.sync_copy(z128_hbm.at[pl.ds(s * STRIPE, STRIPE), :],
                    acc.at[pl.ds(s * STRIPE, STRIPE), :])
    plsc.subcore_barrier()

    iota = lax.iota(i32, 16)
    id8 = iota // 8
    im8 = iota % 8
    izero = iota * 0
    off = c * NP

    @pl.loop(0, NBLK // 16)
    def _blk(k):
        g = s * (NBLK // 16) + k
        pltpu.sync_copy(row_hbm.at[pl.ds(g, 1), :], rowb)
        pltpu.sync_copy(col_hbm.at[pl.ds(g, 1), :], colb)
        pltpu.sync_copy(ex_hbm.at[pl.ds(g * 1024, 1024)], exbuf)

        @pl.loop(0, 8)
        def _shift(j):
            colb2[0, pl.ds(16 * j, 16)] = colb[0, pl.ds(16 * j, 16)] + off

        pltpu.sync_copy(rden_hbm.at[rowb.at[0]], rdbuf)
        pltpu.sync_copy(h2_hbm.at[colb2.at[0]], hbuf)

        @pl.loop(0, 64)
        def _al(v):
            ir = 2 * v + id8
            rdv = plsc.load_gather(rdbuf, [ir, im8])
            albuf[pl.ds(16 * v, 16)] = exbuf[pl.ds(16 * v, 16)] * rdv

        @pl.loop(0, 128)
        def _edge(i):
            base = 8 * i + 4 * c
            for jj in range(4):
                av = plsc.load_gather(albuf, [(base + jj) + izero])
                lo = 32 * jj
                hbuf[i, pl.ds(lo, 16)] = hbuf[i, pl.ds(lo, 16)] * av
                hbuf[i, pl.ds(lo + 16, 16)] = hbuf[i, pl.ds(lo + 16, 16)] * av

        pltpu.sync_copy(hbuf, acc.at[rowb.at[0]], add=True)

    plsc.subcore_barrier()
    pltpu.sync_copy(acc.at[pl.ds(s * STRIPE, STRIPE), :],
                    agg_hbm.at[c].at[pl.ds(s * STRIPE, STRIPE), :])


def _k_agg(row2d, col2d, ex_flat, rden, h2flat, z128):
    return pl.kernel(
        _agg_body,
        out_type=jax.ShapeDtypeStruct((2, NP, 128), f32),
        mesh=_MESH,
        compiler_params=_SC_PARAMS,
        scratch_types=[
            pltpu.VMEM((1, 128), i32),
            pltpu.VMEM((1, 128), i32),
            pltpu.VMEM((1, 128), i32),
            pltpu.VMEM((128, 128), f32),
            pltpu.VMEM((128, 8), f32),
            pltpu.VMEM((1024,), f32),
            pltpu.VMEM((1024,), f32),
            pltpu.VMEM_SHARED((NP, 128), f32),
        ],
    )(row2d, col2d, ex_flat, rden, h2flat, z128)


@jax.jit
def kernel(feats, edge_index, edge_attr, W_fc, W_edge, att, bias, W_out, b_out):
    row = edge_index[:, 0]
    col = edge_index[:, 1]
    rowp = jnp.concatenate([row, jnp.full((EP - E,), N, i32)]).reshape(NBLK, 128)
    colp = jnp.concatenate([col, jnp.full((EP - E,), N, i32)]).reshape(NBLK, 128)

    att_f = att[..., 0]
    r256 = jnp.arange(256)
    a12 = (
        jnp.zeros((256, 16), f32)
        .at[r256, r256 // 32].set(att_f[:, :32].reshape(-1))
        .at[r256, 8 + r256 // 32].set(att_f[:, 32:64].reshape(-1))
    )
    r16 = jnp.arange(16)
    b8 = jnp.zeros((16, 8), f32).at[r16, r16 // 2].set(att_f[:, 64:66].reshape(-1))

    h2, src_tab = _k_node(feats, W_fc.T, a12)
    ea16, t8 = _k_edge(edge_attr, W_edge.T, b8)
    tp = jnp.pad(t8.reshape(-1), (0, (EP - E) * 8))

    z8 = jnp.zeros((NP, 8), f32)
    z128 = jnp.zeros((NP, 128), f32)

    ex_flat, dpart = _k_attn(rowp, colp, tp, src_tab, z8)
    rden = _k_rden(dpart.reshape(2, NP * 8 // 128, 128)).reshape(NP, 8)
    agg2 = _k_agg(rowp, colp, ex_flat, rden, h2.reshape(2 * NP, 128), z128)

    out = _k_out(
        agg2[0], agg2[1],
        W_out[:, :128].T, W_out[:, 128:].T,
        bias[:128].reshape(1, 128), bias[128:].reshape(1, 128),
        b_out.reshape(1, 256),
    )
    return out, edge_index, ea16
